# Initial kernel scaffold; baseline (speedup 1.0000x reference)
import jax
import jax.numpy as jnp
from jax.experimental import pallas as pl


def kernel(features, edge_index, W1, b1, gamma, beta, W2, b2):
    raise NotImplementedError("write your pallas kernel here")


# speedup vs baseline: 35.6942x; 35.6942x over previous
import functools

import jax
import jax.numpy as jnp
from jax import lax
from jax.experimental import pallas as pl
from jax.experimental.pallas import tpu as pltpu
from jax.experimental.pallas import tpu_sc as plsc

_N = 10000
_E = 320000
_D = 128

_NC = 2
_NS = 16
_NW = _NC * _NS
_EPW = _E // _NW
_LANES = 16


def _sc_counts(col_i32):
    mesh = plsc.VectorSubcoreMesh(core_axis_name="c", subcore_axis_name="s")

    @functools.partial(
        pl.kernel,
        mesh=mesh,
        out_type=jax.ShapeDtypeStruct((_NW, _N), jnp.float32),
        scratch_types=[
            pltpu.VMEM((_EPW,), jnp.int32),
            pltpu.VMEM((_N,), jnp.float32),
        ],
        compiler_params=pltpu.CompilerParams(needs_layout_passes=False),
    )
    def k(col_hbm, out_hbm, idx_v, cnt_v):
        wid = lax.axis_index("s") * _NC + lax.axis_index("c")
        base = wid * _EPW
        pltpu.sync_copy(col_hbm.at[pl.ds(base, _EPW)], idx_v)

        zeros = jnp.zeros((_LANES,), jnp.float32)

        def zbody(i, carry):
            cnt_v[pl.ds(i * _LANES, _LANES)] = zeros
            return carry

        lax.fori_loop(0, _N // _LANES, zbody, 0)

        ones = jnp.ones((_LANES,), jnp.float32)

        def body(i, carry):
            idx = idx_v[pl.ds(i * _LANES, _LANES)]
            plsc.addupdate_scatter(cnt_v, [idx], ones)
            return carry

        lax.fori_loop(0, _EPW // _LANES, body, 0)

        pltpu.sync_copy(cnt_v, out_hbm.at[wid])

    return k(col_i32)


def _tc_layer1(features, W1, gamma2d, beta2d):
    def body(x_ref, w_ref, g_ref, b_ref, o_ref):
        x = x_ref[...]
        wa = w_ref[:, :_D]
        x1 = lax.dot_general(
            x, wa, (((1,), (1,)), ((), ())),
            preferred_element_type=jnp.float32,
        )
        mean = jnp.mean(x1, axis=0, keepdims=True)
        var = jnp.mean(x1 * x1, axis=0, keepdims=True) - mean * mean
        inv = lax.rsqrt(var + 1e-5)
        y = (x1 - mean) * inv * g_ref[...] + b_ref[...]
        o_ref[...] = jnp.maximum(y, 0.0)

    return pl.pallas_call(
        body,
        out_shape=jax.ShapeDtypeStruct((_N, _D), jnp.float32),
    )(features, W1, gamma2d, beta2d)


def _tc_layer2(x2, counts_parts, W2, b2_2d):
    def body(x_ref, c_ref, w_ref, b_ref, o_ref):
        x = x_ref[...]
        counts = jnp.sum(c_ref[...], axis=0, keepdims=True)
        m2 = lax.dot_general(
            counts, x, (((1,), (0,)), ((), ())),
            preferred_element_type=jnp.float32,
        ) * (1.0 / _E)
        wa = w_ref[:, :_D]
        wb = w_ref[:, _D:]
        c2 = lax.dot_general(
            m2, wb, (((1,), (1,)), ((), ())),
            preferred_element_type=jnp.float32,
        ) + b_ref[...]
        y = lax.dot_general(
            x, wa, (((1,), (1,)), ((), ())),
            preferred_element_type=jnp.float32,
        )
        o_ref[...] = y + c2

    return pl.pallas_call(
        body,
        out_shape=jax.ShapeDtypeStruct((_N, _D), jnp.float32),
    )(x2, counts_parts, W2, b2_2d)


@jax.jit
def kernel(features, edge_index, W1, b1, gamma, beta, W2, b2):
    col = edge_index[1].astype(jnp.int32)
    counts_parts = _sc_counts(col)
    x2 = _tc_layer1(
        features, W1, gamma.reshape(1, _D), beta.reshape(1, _D)
    )
    return _tc_layer2(x2, counts_parts, W2, b2.reshape(1, _D))


# speedup vs baseline: 36.8146x; 1.0314x over previous
import functools

import jax
import jax.numpy as jnp
from jax import lax
from jax.experimental import pallas as pl
from jax.experimental.pallas import tpu as pltpu
from jax.experimental.pallas import tpu_sc as plsc

_N = 10000
_E = 320000
_D = 128

_NC = 2
_NS = 16
_NW = _NC * _NS
_EPW = _E // _NW
_LANES = 16


def _sc_counts(col_i32):
    mesh = plsc.VectorSubcoreMesh(core_axis_name="c", subcore_axis_name="s")

    @functools.partial(
        pl.kernel,
        mesh=mesh,
        out_type=jax.ShapeDtypeStruct((_NW, _N), jnp.float32),
        scratch_types=[
            pltpu.VMEM((_EPW,), jnp.int32),
            pltpu.VMEM((_N,), jnp.float32),
            pltpu.SemaphoreType.DMA,
        ],
        compiler_params=pltpu.CompilerParams(needs_layout_passes=False),
    )
    def k(col_hbm, out_hbm, idx_v, cnt_v, sem):
        wid = lax.axis_index("s") * _NC + lax.axis_index("c")
        base = wid * _EPW
        copy = pltpu.make_async_copy(col_hbm.at[pl.ds(base, _EPW)], idx_v, sem)
        copy.start()

        unroll = 25
        zeros = jnp.zeros((_LANES,), jnp.float32)

        def zbody(i, carry):
            zb = i * (_LANES * unroll)
            for j in range(unroll):
                cnt_v[pl.ds(zb + j * _LANES, _LANES)] = zeros
            return carry

        lax.fori_loop(0, _N // (_LANES * unroll), zbody, 0)
        copy.wait()

        ones = jnp.ones((_LANES,), jnp.float32)

        def body(i, carry):
            cb = i * (_LANES * unroll)
            for j in range(unroll):
                idx = idx_v[pl.ds(cb + j * _LANES, _LANES)]
                plsc.addupdate_scatter(cnt_v, [idx], ones)
            return carry

        lax.fori_loop(0, _EPW // (_LANES * unroll), body, 0)

        pltpu.sync_copy(cnt_v, out_hbm.at[wid])

    return k(col_i32)


def _tc_layer1(features, W1, gamma2d, beta2d):
    def body(x_ref, w_ref, g_ref, b_ref, o_ref):
        x = x_ref[...]
        wa = w_ref[:, :_D]
        x1 = lax.dot_general(
            x, wa, (((1,), (1,)), ((), ())),
            preferred_element_type=jnp.float32,
        )
        mean = jnp.mean(x1, axis=0, keepdims=True)
        var = jnp.mean(x1 * x1, axis=0, keepdims=True) - mean * mean
        inv = lax.rsqrt(var + 1e-5)
        y = (x1 - mean) * inv * g_ref[...] + b_ref[...]
        o_ref[...] = jnp.maximum(y, 0.0)

    return pl.pallas_call(
        body,
        out_shape=jax.ShapeDtypeStruct((_N, _D), jnp.float32),
    )(features, W1, gamma2d, beta2d)


def _tc_layer2(x2, counts_parts, W2, b2_2d):
    def body(x_ref, c_ref, w_ref, b_ref, o_ref):
        x = x_ref[...]
        counts = jnp.sum(c_ref[...], axis=0, keepdims=True)
        m2 = lax.dot_general(
            counts, x, (((1,), (0,)), ((), ())),
            preferred_element_type=jnp.float32,
        ) * (1.0 / _E)
        wa = w_ref[:, :_D]
        wb = w_ref[:, _D:]
        c2 = lax.dot_general(
            m2, wb, (((1,), (1,)), ((), ())),
            preferred_element_type=jnp.float32,
        ) + b_ref[...]
        y = lax.dot_general(
            x, wa, (((1,), (1,)), ((), ())),
            preferred_element_type=jnp.float32,
        )
        o_ref[...] = y + c2

    return pl.pallas_call(
        body,
        out_shape=jax.ShapeDtypeStruct((_N, _D), jnp.float32),
    )(x2, counts_parts, W2, b2_2d)


@jax.jit
def kernel(features, edge_index, W1, b1, gamma, beta, W2, b2):
    col = edge_index[1].astype(jnp.int32)
    counts_parts = _sc_counts(col)
    x2 = _tc_layer1(
        features, W1, gamma.reshape(1, _D), beta.reshape(1, _D)
    )
    return _tc_layer2(x2, counts_parts, W2, b2.reshape(1, _D))


# speedup vs baseline: 52.1968x; 1.4178x over previous
import functools

import jax
import jax.numpy as jnp
from jax import lax
from jax.experimental import pallas as pl
from jax.experimental.pallas import tpu as pltpu
from jax.experimental.pallas import tpu_sc as plsc

_N = 10000
_E = 320000
_D = 128

_NC = 2
_NS = 16
_NW = _NC * _NS
_EPW = _E // _NW
_LANES = 16


_NCH = _E // 128
_T = -(-_NCH // _NW)


def _sc_counts(edge_index_i32):
    mesh = plsc.VectorSubcoreMesh(core_axis_name="c", subcore_axis_name="s")

    @functools.partial(
        pl.kernel,
        mesh=mesh,
        out_type=jax.ShapeDtypeStruct((_NW, _N), jnp.float32),
        scratch_types=[
            pltpu.VMEM((2, _T * 128), jnp.int32),
            pltpu.VMEM((_N,), jnp.float32),
            pltpu.SemaphoreType.DMA,
        ],
        compiler_params=pltpu.CompilerParams(needs_layout_passes=False),
    )
    def k(col_hbm, out_hbm, idx_v, cnt_v, sem):
        wid = lax.axis_index("s") * _NC + lax.axis_index("c")
        owned0 = wid * _T
        base_ch = jnp.minimum(owned0, _NCH - _T)
        copy = pltpu.make_async_copy(
            col_hbm.at[:, pl.ds(base_ch * 128, _T * 128)], idx_v, sem
        )
        copy.start()

        unroll = 25
        zeros = jnp.zeros((_LANES,), jnp.float32)

        def zbody(i, carry):
            zb = i * (_LANES * unroll)
            for j in range(unroll):
                cnt_v[pl.ds(zb + j * _LANES, _LANES)] = zeros
            return carry

        lax.fori_loop(0, _N // (_LANES * unroll), zbody, 0)
        copy.wait()

        ones = jnp.ones((_LANES,), jnp.float32)

        def body(j, carry):
            mask = jnp.full((_LANES,), base_ch + j >= owned0)
            for kk in range(128 // _LANES):
                idx = idx_v[1, pl.ds(j * 128 + kk * _LANES, _LANES)]
                plsc.addupdate_scatter(cnt_v, [idx], ones, mask=mask)
            return carry

        lax.fori_loop(0, _T, body, 0)

        pltpu.sync_copy(cnt_v, out_hbm.at[wid])

    return k(edge_index_i32)


def _tc_layer1(features, W1, gamma2d, beta2d):
    def body(x_ref, w_ref, g_ref, b_ref, o_ref):
        x = x_ref[...]
        wa = w_ref[:, :_D]
        x1 = lax.dot_general(
            x, wa, (((1,), (1,)), ((), ())),
            preferred_element_type=jnp.float32,
        )
        mean = jnp.mean(x1, axis=0, keepdims=True)
        var = jnp.mean(x1 * x1, axis=0, keepdims=True) - mean * mean
        inv = lax.rsqrt(var + 1e-5)
        y = (x1 - mean) * inv * g_ref[...] + b_ref[...]
        o_ref[...] = jnp.maximum(y, 0.0)

    return pl.pallas_call(
        body,
        out_shape=jax.ShapeDtypeStruct((_N, _D), jnp.float32),
    )(features, W1, gamma2d, beta2d)


def _tc_layer2(x2, counts_parts, W2, b2_2d):
    def body(x_ref, c_ref, w_ref, b_ref, o_ref):
        x = x_ref[...]
        counts = jnp.sum(c_ref[...], axis=0, keepdims=True)
        m2 = lax.dot_general(
            counts, x, (((1,), (0,)), ((), ())),
            preferred_element_type=jnp.float32,
        ) * (1.0 / _E)
        wa = w_ref[:, :_D]
        wb = w_ref[:, _D:]
        c2 = lax.dot_general(
            m2, wb, (((1,), (1,)), ((), ())),
            preferred_element_type=jnp.float32,
        ) + b_ref[...]
        y = lax.dot_general(
            x, wa, (((1,), (1,)), ((), ())),
            preferred_element_type=jnp.float32,
        )
        o_ref[...] = y + c2

    return pl.pallas_call(
        body,
        out_shape=jax.ShapeDtypeStruct((_N, _D), jnp.float32),
    )(x2, counts_parts, W2, b2_2d)


@jax.jit
def kernel(features, edge_index, W1, b1, gamma, beta, W2, b2):
    if edge_index.dtype != jnp.int32:
        edge_index = edge_index.astype(jnp.int32)
    counts_parts = _sc_counts(edge_index)
    x2 = _tc_layer1(
        features, W1, gamma.reshape(1, _D), beta.reshape(1, _D)
    )
    return _tc_layer2(x2, counts_parts, W2, b2.reshape(1, _D))


# speedup vs baseline: 52.3367x; 1.0027x over previous
import functools

import jax
import jax.numpy as jnp
from jax import lax
from jax.experimental import pallas as pl
from jax.experimental.pallas import tpu as pltpu
from jax.experimental.pallas import tpu_sc as plsc

_N = 10000
_E = 320000
_D = 128

_NC = 2
_NS = 16
_NW = _NC * _NS
_EPW = _E // _NW
_LANES = 16


_NCH = _E // 128
_T = -(-_NCH // _NW)


def _sc_counts(edge_index_i32):
    mesh = plsc.VectorSubcoreMesh(core_axis_name="c", subcore_axis_name="s")

    @functools.partial(
        pl.kernel,
        mesh=mesh,
        out_type=jax.ShapeDtypeStruct((_NW, _N), jnp.float32),
        scratch_types=[
            pltpu.VMEM((2, _T * 128), jnp.int32),
            pltpu.VMEM((_N,), jnp.float32),
            pltpu.SemaphoreType.DMA,
        ],
        compiler_params=pltpu.CompilerParams(needs_layout_passes=False),
    )
    def k(col_hbm, out_hbm, idx_v, cnt_v, sem):
        wid = lax.axis_index("s") * _NC + lax.axis_index("c")
        owned0 = wid * _T
        base_ch = jnp.minimum(owned0, _NCH - _T)
        skip = owned0 - base_ch
        nch = jnp.minimum(_T, _NCH - owned0)
        copy = pltpu.make_async_copy(
            col_hbm.at[:, pl.ds(base_ch * 128, _T * 128)], idx_v, sem
        )
        copy.start()

        unroll = 25
        zeros = jnp.zeros((_LANES,), jnp.float32)

        def zbody(i, carry):
            zb = i * (_LANES * unroll)
            for j in range(unroll):
                cnt_v[pl.ds(zb + j * _LANES, _LANES)] = zeros
            return carry

        lax.fori_loop(0, _N // (_LANES * unroll), zbody, 0)
        copy.wait()

        ones = jnp.ones((_LANES,), jnp.float32)

        def body(j, carry):
            cb = (skip + j) * 128
            for kk in range(128 // _LANES):
                idx = idx_v[1, pl.ds(cb + kk * _LANES, _LANES)]
                plsc.addupdate_scatter(cnt_v, [idx], ones)
            return carry

        lax.fori_loop(0, nch, body, 0)

        pltpu.sync_copy(cnt_v, out_hbm.at[wid])

    return k(edge_index_i32)


def _tc_layer1(features, W1, gamma2d, beta2d):
    def body(x_ref, w_ref, g_ref, b_ref, o_ref):
        x = x_ref[...]
        wa = w_ref[:, :_D]
        x1 = lax.dot_general(
            x, wa, (((1,), (1,)), ((), ())),
            preferred_element_type=jnp.float32,
        )
        mean = jnp.mean(x1, axis=0, keepdims=True)
        var = jnp.mean(x1 * x1, axis=0, keepdims=True) - mean * mean
        inv = lax.rsqrt(var + 1e-5)
        y = (x1 - mean) * inv * g_ref[...] + b_ref[...]
        o_ref[...] = jnp.maximum(y, 0.0).astype(jnp.bfloat16)

    return pl.pallas_call(
        body,
        out_shape=jax.ShapeDtypeStruct((_N, _D), jnp.bfloat16),
    )(features, W1, gamma2d, beta2d)


def _tc_layer2(x2, counts_parts, W2, b2_2d):
    def body(x_ref, c_ref, w_ref, b_ref, o_ref):
        x = x_ref[...]
        counts = jnp.sum(c_ref[...], axis=0, keepdims=True)
        m2 = lax.dot_general(
            counts.astype(jnp.bfloat16), x, (((1,), (0,)), ((), ())),
            preferred_element_type=jnp.float32,
        ) * (1.0 / _E)
        wa = w_ref[:, :_D]
        wb = w_ref[:, _D:]
        c2 = lax.dot_general(
            m2, wb, (((1,), (1,)), ((), ())),
            preferred_element_type=jnp.float32,
        ) + b_ref[...]
        y = lax.dot_general(
            x, wa.astype(jnp.bfloat16), (((1,), (1,)), ((), ())),
            preferred_element_type=jnp.float32,
        )
        o_ref[...] = y + c2

    return pl.pallas_call(
        body,
        out_shape=jax.ShapeDtypeStruct((_N, _D), jnp.float32),
    )(x2, counts_parts, W2, b2_2d)


@jax.jit
def kernel(features, edge_index, W1, b1, gamma, beta, W2, b2):
    if edge_index.dtype != jnp.int32:
        edge_index = edge_index.astype(jnp.int32)
    counts_parts = _sc_counts(edge_index)
    x2 = _tc_layer1(
        features, W1, gamma.reshape(1, _D), beta.reshape(1, _D)
    )
    return _tc_layer2(x2, counts_parts, W2, b2.reshape(1, _D))


# speedup vs baseline: 52.4301x; 1.0018x over previous
import functools

import jax
import jax.numpy as jnp
from jax import lax
from jax.experimental import pallas as pl
from jax.experimental.pallas import tpu as pltpu
from jax.experimental.pallas import tpu_sc as plsc

_N = 10000
_E = 320000
_D = 128

_NC = 2
_NS = 16
_NW = _NC * _NS
_EPW = _E // _NW
_LANES = 16


_NCH = _E // 128
_T = -(-_NCH // _NW)


def _sc_counts(edge_index_i32):
    mesh = plsc.VectorSubcoreMesh(core_axis_name="c", subcore_axis_name="s")

    @functools.partial(
        pl.kernel,
        mesh=mesh,
        out_type=jax.ShapeDtypeStruct((_NW, _N), jnp.float32),
        scratch_types=[
            pltpu.VMEM((2, _T * 128), jnp.int32),
            pltpu.VMEM((_N,), jnp.float32),
            pltpu.SemaphoreType.DMA,
        ],
        compiler_params=pltpu.CompilerParams(
            needs_layout_passes=False, skip_device_barrier=True
        ),
    )
    def k(col_hbm, out_hbm, idx_v, cnt_v, sem):
        wid = lax.axis_index("s") * _NC + lax.axis_index("c")
        owned0 = wid * _T
        base_ch = jnp.minimum(owned0, _NCH - _T)
        skip = owned0 - base_ch
        nch = jnp.minimum(_T, _NCH - owned0)
        copy = pltpu.make_async_copy(
            col_hbm.at[:, pl.ds(base_ch * 128, _T * 128)], idx_v, sem
        )
        copy.start()

        unroll = 25
        zeros = jnp.zeros((_LANES,), jnp.float32)

        def zbody(i, carry):
            zb = i * (_LANES * unroll)
            for j in range(unroll):
                cnt_v[pl.ds(zb + j * _LANES, _LANES)] = zeros
            return carry

        lax.fori_loop(0, _N // (_LANES * unroll), zbody, 0)
        copy.wait()

        ones = jnp.ones((_LANES,), jnp.float32)

        def body(j, carry):
            cb = (skip + j) * 128
            for kk in range(128 // _LANES):
                idx = idx_v[1, pl.ds(cb + kk * _LANES, _LANES)]
                plsc.addupdate_scatter(cnt_v, [idx], ones)
            return carry

        lax.fori_loop(0, nch, body, 0)

        pltpu.sync_copy(cnt_v, out_hbm.at[wid])

    return k(edge_index_i32)


def _tc_layer1(features, W1, gamma2d, beta2d):
    def body(x_ref, w_ref, g_ref, b_ref, o_ref):
        x = x_ref[...]
        wa = w_ref[:, :_D]
        x1 = lax.dot_general(
            x, wa, (((1,), (1,)), ((), ())),
            preferred_element_type=jnp.float32,
        )
        mean = jnp.mean(x1, axis=0, keepdims=True)
        var = jnp.mean(x1 * x1, axis=0, keepdims=True) - mean * mean
        inv = lax.rsqrt(var + 1e-5)
        y = (x1 - mean) * inv * g_ref[...] + b_ref[...]
        o_ref[...] = jnp.maximum(y, 0.0).astype(jnp.bfloat16)

    return pl.pallas_call(
        body,
        out_shape=jax.ShapeDtypeStruct((_N, _D), jnp.bfloat16),
    )(features, W1, gamma2d, beta2d)


def _tc_layer2(x2, counts_parts, W2, b2_2d):
    def body(x_ref, c_ref, w_ref, b_ref, o_ref):
        x = x_ref[...]
        counts = jnp.sum(c_ref[...], axis=0, keepdims=True)
        m2 = lax.dot_general(
            counts.astype(jnp.bfloat16), x, (((1,), (0,)), ((), ())),
            preferred_element_type=jnp.float32,
        ) * (1.0 / _E)
        wa = w_ref[:, :_D]
        wb = w_ref[:, _D:]
        c2 = lax.dot_general(
            m2, wb, (((1,), (1,)), ((), ())),
            preferred_element_type=jnp.float32,
        ) + b_ref[...]
        y = lax.dot_general(
            x, wa.astype(jnp.bfloat16), (((1,), (1,)), ((), ())),
            preferred_element_type=jnp.float32,
        )
        o_ref[...] = y + c2

    return pl.pallas_call(
        body,
        out_shape=jax.ShapeDtypeStruct((_N, _D), jnp.float32),
    )(x2, counts_parts, W2, b2_2d)


@jax.jit
def kernel(features, edge_index, W1, b1, gamma, beta, W2, b2):
    if edge_index.dtype != jnp.int32:
        edge_index = edge_index.astype(jnp.int32)
    counts_parts = _sc_counts(edge_index)
    x2 = _tc_layer1(
        features, W1, gamma.reshape(1, _D), beta.reshape(1, _D)
    )
    return _tc_layer2(x2, counts_parts, W2, b2.reshape(1, _D))


# speedup vs baseline: 52.6240x; 1.0037x over previous
import functools

import jax
import jax.numpy as jnp
from jax import lax
from jax.experimental import pallas as pl
from jax.experimental.pallas import tpu as pltpu
from jax.experimental.pallas import tpu_sc as plsc

_N = 10000
_E = 320000
_D = 128

_NC = 2
_NS = 16
_NW = _NC * _NS
_EPW = _E // _NW
_LANES = 16


_NCH = _E // 128
_T = -(-_NCH // _NW)


def _sc_counts(edge_index_i32):
    mesh = plsc.VectorSubcoreMesh(core_axis_name="c", subcore_axis_name="s")

    @functools.partial(
        pl.kernel,
        mesh=mesh,
        out_type=jax.ShapeDtypeStruct((_NW, _N), jnp.float32),
        scratch_types=[
            pltpu.VMEM((2, _T * 128), jnp.int32),
            pltpu.VMEM((_N,), jnp.float32),
            pltpu.SemaphoreType.DMA,
        ],
        compiler_params=pltpu.CompilerParams(needs_layout_passes=False),
    )
    def k(col_hbm, out_hbm, idx_v, cnt_v, sem):
        wid = lax.axis_index("s") * _NC + lax.axis_index("c")
        owned0 = wid * _T
        base_ch = jnp.minimum(owned0, _NCH - _T)
        skip = owned0 - base_ch
        nch = jnp.minimum(_T, _NCH - owned0)
        copy = pltpu.make_async_copy(
            col_hbm.at[:, pl.ds(base_ch * 128, _T * 128)], idx_v, sem
        )
        copy.start()

        unroll = 25
        zeros = jnp.zeros((_LANES,), jnp.float32)

        def zbody(i, carry):
            zb = i * (_LANES * unroll)
            for j in range(unroll):
                cnt_v[pl.ds(zb + j * _LANES, _LANES)] = zeros
            return carry

        lax.fori_loop(0, _N // (_LANES * unroll), zbody, 0)
        copy.wait()

        ones = jnp.ones((_LANES,), jnp.float32)

        def scatter_chunk(cb):
            for kk in range(128 // _LANES):
                idx = idx_v[1, pl.ds(cb + kk * _LANES, _LANES)]
                plsc.addupdate_scatter(cnt_v, [idx], ones)

        def body(j, carry):
            cb = (skip + 2 * j) * 128
            scatter_chunk(cb)
            scatter_chunk(cb + 128)
            return carry

        lax.fori_loop(0, nch // 2, body, 0)
        scatter_chunk((skip + nch - 1) * 128)

        pltpu.sync_copy(cnt_v, out_hbm.at[wid])

    return k(edge_index_i32)


def _tc_layer1(features, W1, gamma2d, beta2d):
    def body(x_ref, w_ref, g_ref, b_ref, o_ref):
        x = x_ref[...].astype(jnp.bfloat16)
        wa = w_ref[:, :_D].astype(jnp.bfloat16)
        x1 = lax.dot_general(
            x, wa, (((1,), (1,)), ((), ())),
            preferred_element_type=jnp.float32,
        )
        mean = jnp.mean(x1, axis=0, keepdims=True)
        var = jnp.mean(x1 * x1, axis=0, keepdims=True) - mean * mean
        inv = lax.rsqrt(var + 1e-5)
        y = (x1 - mean) * inv * g_ref[...] + b_ref[...]
        o_ref[...] = jnp.maximum(y, 0.0).astype(jnp.bfloat16)

    return pl.pallas_call(
        body,
        out_shape=jax.ShapeDtypeStruct((_N, _D), jnp.bfloat16),
    )(features, W1, gamma2d, beta2d)


def _tc_layer2(x2, counts_parts, W2, b2_2d):
    def body(x_ref, c_ref, w_ref, b_ref, o_ref):
        x = x_ref[...]
        counts = jnp.sum(c_ref[...], axis=0, keepdims=True)
        m2 = lax.dot_general(
            counts.astype(jnp.bfloat16), x, (((1,), (0,)), ((), ())),
            preferred_element_type=jnp.float32,
        ) * (1.0 / _E)
        wa = w_ref[:, :_D]
        wb = w_ref[:, _D:]
        c2 = lax.dot_general(
            m2, wb, (((1,), (1,)), ((), ())),
            preferred_element_type=jnp.float32,
        ) + b_ref[...]
        y = lax.dot_general(
            x, wa.astype(jnp.bfloat16), (((1,), (1,)), ((), ())),
            preferred_element_type=jnp.float32,
        )
        o_ref[...] = y + c2

    return pl.pallas_call(
        body,
        out_shape=jax.ShapeDtypeStruct((_N, _D), jnp.float32),
    )(x2, counts_parts, W2, b2_2d)


@jax.jit
def kernel(features, edge_index, W1, b1, gamma, beta, W2, b2):
    if edge_index.dtype != jnp.int32:
        edge_index = edge_index.astype(jnp.int32)
    counts_parts = _sc_counts(edge_index)
    x2 = _tc_layer1(
        features, W1, gamma.reshape(1, _D), beta.reshape(1, _D)
    )
    return _tc_layer2(x2, counts_parts, W2, b2.reshape(1, _D))
